# baseline (speedup 1.0000x reference)
import functools
import jax
import jax.numpy as jnp
from jax import lax
from jax.experimental import pallas as pl
from jax.experimental.pallas import tpu as pltpu
from jax.experimental.pallas import tpu_sc as plsc

D = 64
SCALE = 8.0
NC, NS, L = 2, 16, 16
NW = NC * NS
B = 4096 * 200
BPW = B // NW
C = 512
NCHUNK = BPW // C

_mesh = plsc.VectorSubcoreMesh(core_axis_name="c", subcore_axis_name="s")


@functools.partial(
    pl.kernel,
    mesh=_mesh,
    out_type=jax.ShapeDtypeStruct((B, D), jnp.float32),
    scratch_types=[
        pltpu.VMEM((C,), jnp.int32),
        pltpu.VMEM((C, D), jnp.float32),
        pltpu.SemaphoreType.DMA,
    ],
    compiler_params=pltpu.CompilerParams(use_tc_tiling_on_sc=False),
)
def _emb_lookup(idx_hbm, table_hbm, out_hbm, idx_c, rows, sem):
    wid = lax.axis_index("s") * NC + lax.axis_index("c")
    base = wid * BPW

    def chunk_body(g, carry):
        pltpu.sync_copy(idx_hbm.at[wid, g], idx_c)
        pltpu.async_copy(table_hbm.at[idx_c], rows, sem).wait()

        def row_body(i, c2):
            for j in range(D // L):
                rows[i, pl.ds(j * L, L)] = rows[i, pl.ds(j * L, L)] * SCALE
            return c2

        lax.fori_loop(0, C, row_body, 0, unroll=2)
        pltpu.sync_copy(rows, out_hbm.at[pl.ds(base + g * C, C)])
        return carry

    lax.fori_loop(0, NCHUNK, chunk_body, 0)


def kernel(tokens, table):
    idx = tokens.astype(jnp.int32).reshape(NW, NCHUNK, C)
    out = _emb_lookup(idx, table)
    return out.reshape(tokens.shape[0], tokens.shape[1], D)


# speedup vs baseline: 1.0892x; 1.0892x over previous
import functools
import jax
import jax.numpy as jnp
from jax import lax
from jax.experimental import pallas as pl
from jax.experimental.pallas import tpu as pltpu
from jax.experimental.pallas import tpu_sc as plsc

D = 64
SCALE = 8.0
NC, NS, L = 2, 16, 16
NW = NC * NS
B = 4096 * 200
BPW = B // NW
C = 256
NCHUNK = BPW // C
NBUF = 4
LOOKAHEAD = 2

_mesh = plsc.VectorSubcoreMesh(core_axis_name="c", subcore_axis_name="s")


@functools.partial(
    pl.kernel,
    mesh=_mesh,
    out_type=jax.ShapeDtypeStruct((B, D), jnp.float32),
    scratch_types=[
        pltpu.VMEM((NCHUNK, C), jnp.int32),
        [pltpu.VMEM((C, D), jnp.float32) for _ in range(NBUF)],
        [pltpu.SemaphoreType.DMA for _ in range(NBUF)],
        [pltpu.SemaphoreType.DMA for _ in range(NBUF)],
    ],
    compiler_params=pltpu.CompilerParams(use_tc_tiling_on_sc=False),
)
def _emb_lookup(idx_hbm, table_hbm, out_hbm, idx_all, rows, gsem, wsem):
    wid = lax.axis_index("s") * NC + lax.axis_index("c")
    base = wid * BPW
    pltpu.sync_copy(idx_hbm.at[wid], idx_all)

    def gather_wait(b):
        pltpu.make_async_copy(table_hbm.at[pl.ds(0, C)], rows[b], gsem[b]).wait()

    def write_wait(b):
        pltpu.make_async_copy(table_hbm.at[pl.ds(0, C)], rows[b], wsem[b]).wait()

    for g in range(LOOKAHEAD):
        pltpu.async_copy(table_hbm.at[idx_all.at[g]], rows[g], gsem[g])

    def outer(i, carry):
        gbase = i * NBUF
        for b in range(NBUF):
            g = gbase + b
            gather_wait(b)

            def row_body(r, c2):
                for j in range(D // L):
                    rows[b][r, pl.ds(j * L, L)] = rows[b][r, pl.ds(j * L, L)] * SCALE
                return c2

            lax.fori_loop(0, C, row_body, 0, unroll=4)
            pltpu.async_copy(rows[b], out_hbm.at[pl.ds(base + g * C, C)], wsem[b])

            g2 = g + LOOKAHEAD
            b2 = (b + LOOKAHEAD) % NBUF

            @pl.when(g2 < NCHUNK)
            def _():
                @pl.when(g2 >= NBUF)
                def _():
                    write_wait(b2)

                pltpu.async_copy(table_hbm.at[idx_all.at[g2]], rows[b2], gsem[b2])

        return carry

    lax.fori_loop(0, NCHUNK // NBUF, outer, 0)
    for b in range(NBUF):
        write_wait(b)


def kernel(tokens, table):
    idx = tokens.astype(jnp.int32).reshape(NW, NCHUNK, C)
    out = _emb_lookup(idx, table)
    return out.reshape(tokens.shape[0], tokens.shape[1], D)
